# Initial kernel scaffold; baseline (speedup 1.0000x reference)
import jax
import jax.numpy as jnp
from jax.experimental import pallas as pl


def kernel(pos, edge_index, W1, b1, W2, b2, W3, b3):
    raise NotImplementedError("write your pallas kernel here")


# speedup vs baseline: 10.0078x; 10.0078x over previous
import functools

import jax
import jax.numpy as jnp
from jax import lax
from jax.experimental import pallas as pl
from jax.experimental.pallas import tpu as pltpu
from jax.experimental.pallas import tpu_sc as plsc

N = 50000
E = 800000
N_PAD = 51200
ROWS_PER_TILE = N_PAD // 16
DRAIN_CH = ROWS_PER_TILE // 128
NC, NS = 2, 16
J = 196
E_PAD = NC * NS * J * 128
HR = 512
BLK = 512
GRID = N_PAD // BLK

_mesh = plsc.VectorSubcoreMesh(
    core_axis_name="c", subcore_axis_name="s", num_cores=NC, num_subcores=NS)


@functools.partial(
    pl.kernel,
    out_type=jax.ShapeDtypeStruct((NC * NS, HR * 128), jnp.float32),
    mesh=_mesh,
    scratch_types=[
        pltpu.VMEM((J, 128), jnp.int32),
        pltpu.VMEM((HR * 128,), jnp.float32),
    ],
    compiler_params=pltpu.CompilerParams(needs_layout_passes=False),
)
def _deg_kernel(dst3, zeros1d, histout, dstv, hist):
    c = lax.axis_index("c")
    s = lax.axis_index("s")
    wid = c * NS + s
    pltpu.sync_copy(dst3.at[wid], dstv)
    pltpu.sync_copy(zeros1d, hist)
    ones16 = jnp.full((16,), 1.0, jnp.float32)

    def hbody(j, carry):
        for l in range(8):
            idx = dstv[j, pl.ds(l * 16, 16)]
            plsc.addupdate_scatter(hist, [idx], ones16)
        return carry

    lax.fori_loop(0, J, hbody, 0)
    pltpu.sync_copy(hist, histout.at[wid])


def _make_prop(num_groups, cg):
    @functools.partial(
        pl.kernel,
        out_type=[jax.ShapeDtypeStruct((NC, N_PAD, cg), jnp.float32)
                  for _ in range(num_groups)],
        mesh=_mesh,
        scratch_types=[
            pltpu.VMEM((J, 128), jnp.int32),
            pltpu.VMEM((J, 128), jnp.int32),
            pltpu.VMEM((128, cg), jnp.float32),
            pltpu.VMEM((128, cg), jnp.float32),
            pltpu.VMEM((128, cg), jnp.float32),
            pltpu.VMEM_SHARED((N_PAD, cg), jnp.float32),
            pltpu.SemaphoreType.DMA,
        ],
        compiler_params=pltpu.CompilerParams(
            needs_layout_passes=False, use_tc_tiling_on_sc=False),
    )
    def prop(src3, dst3, zrow, *rest):
        ys = rest[:num_groups]
        outs = rest[num_groups:2 * num_groups]
        srcv, dstv, zbuf, gbuf, dbuf, acc, sem = rest[2 * num_groups:]
        c = lax.axis_index("c")
        s = lax.axis_index("s")
        wid = c * NS + s
        pltpu.sync_copy(src3.at[wid], srcv)
        pltpu.sync_copy(dst3.at[wid], dstv)
        pltpu.sync_copy(zrow, zbuf)
        for g in range(num_groups):
            y = ys[g]
            out = outs[g]

            def zbody(q, carry):
                pltpu.sync_copy(zbuf, acc.at[pl.ds(s * ROWS_PER_TILE + q * 128, 128)])
                return carry

            lax.fori_loop(0, DRAIN_CH, zbody, 0)
            plsc.subcore_barrier()

            def ebody(j, carry):
                pltpu.async_copy(y.at[srcv.at[j]], gbuf, sem).wait()
                pltpu.sync_copy(gbuf, acc.at[dstv.at[j]], add=True)
                return carry

            lax.fori_loop(0, J, ebody, 0)
            plsc.subcore_barrier()

            def dbody(q, carry):
                base = s * ROWS_PER_TILE + q * 128
                pltpu.sync_copy(acc.at[pl.ds(base, 128)], dbuf)
                pltpu.sync_copy(dbuf, out.at[c, pl.ds(base, 128)])
                return carry

            lax.fori_loop(0, DRAIN_CH, dbody, 0)

    return prop


_prop1 = _make_prop(1, 16)
_prop4 = _make_prop(4, 16)


def _row_spec(ch):
    return pl.BlockSpec((BLK, ch), lambda i: (i, 0))


def _full_spec(r, ch):
    return pl.BlockSpec((r, ch), lambda i: (0, 0))


def _k0_body(hs, degr):
    degr[...] = jnp.sum(hs[...], axis=0)


def _k1_body(h, posr, disr, y1r):
    deg = h[...] + 1.0
    dis = lax.rsqrt(deg)
    disr[...] = dis
    y1r[...] = dis * posr[...]


def _k2_body(disr, s0, s1, y1, w, b, *youts):
    dis = disr[...]
    z = dis * (s0[...] + s1[...] + y1[...])
    h = jnp.maximum(
        jnp.dot(z, w[...], preferred_element_type=jnp.float32) + b[...], 0.0)
    y2 = dis * h
    for g in range(4):
        youts[g][...] = y2[:, g * 16:(g + 1) * 16]


def _gather_z(disr, srefs):
    dis = disr[...]
    zs = [dis * (srefs[3 * g][...] + srefs[3 * g + 1][...] + srefs[3 * g + 2][...])
          for g in range(4)]
    return dis, jnp.concatenate(zs, axis=1)


def _k3_body(disr, *rest):
    srefs, (w, b) = rest[:12], rest[12:14]
    youts = rest[14:]
    dis, z = _gather_z(disr, srefs)
    h = jnp.maximum(
        jnp.dot(z, w[...], preferred_element_type=jnp.float32) + b[...], 0.0)
    y3 = dis * h
    for g in range(4):
        youts[g][...] = y3[:, g * 16:(g + 1) * 16]


def _k4_body(disr, *rest):
    srefs, (w, b), outr = rest[:12], rest[12:14], rest[14]
    _, z = _gather_z(disr, srefs)
    t = jnp.dot(z, w[...], preferred_element_type=jnp.float32) + b[...]
    outr[...] = 1.0 / (1.0 + jnp.exp(-t))


def _shape(ch):
    return jax.ShapeDtypeStruct((N_PAD, ch), jnp.float32)


_k0 = pl.pallas_call(
    _k0_body, grid=(4,),
    in_specs=[pl.BlockSpec((NC * NS, 128, 128), lambda i: (0, i, 0))],
    out_specs=pl.BlockSpec((128, 128), lambda i: (i, 0)),
    out_shape=jax.ShapeDtypeStruct((HR, 128), jnp.float32))

_k1 = pl.pallas_call(
    _k1_body, grid=(GRID,),
    in_specs=[_row_spec(1), _row_spec(16)],
    out_specs=[_row_spec(1), _row_spec(16)],
    out_shape=[_shape(1), _shape(16)])

_k2 = pl.pallas_call(
    _k2_body, grid=(GRID,),
    in_specs=[_row_spec(1), _row_spec(16), _row_spec(16), _row_spec(16),
              _full_spec(16, 64), _full_spec(1, 64)],
    out_specs=[_row_spec(16)] * 4,
    out_shape=[_shape(16)] * 4)

_k3 = pl.pallas_call(
    _k3_body, grid=(GRID,),
    in_specs=[_row_spec(1)] + [_row_spec(16)] * 12 +
             [_full_spec(64, 64), _full_spec(1, 64)],
    out_specs=[_row_spec(16)] * 4,
    out_shape=[_shape(16)] * 4)

_k4 = pl.pallas_call(
    _k4_body, grid=(GRID,),
    in_specs=[_row_spec(1)] + [_row_spec(16)] * 12 +
             [_full_spec(64, 68), _full_spec(1, 68)],
    out_specs=_row_spec(68),
    out_shape=_shape(68))


def kernel(pos, edge_index, W1, b1, W2, b2, W3, b3):
    src = edge_index[0]
    dst = edge_index[1]
    pad = jnp.full((E_PAD - E,), N, jnp.int32)
    src3 = jnp.concatenate([src, pad]).reshape(NC * NS, J, 128)
    dst3 = jnp.concatenate([dst, pad]).reshape(NC * NS, J, 128)

    pos_p = jnp.zeros((N_PAD, 16), jnp.float32).at[:N, :3].set(pos)
    w1p = jnp.zeros((16, 64), jnp.float32).at[:3].set(W1)
    b1r = b1.reshape(1, 64)
    b2r = b2.reshape(1, 64)
    b3r = b3.reshape(1, 68)

    zeros1d = jnp.zeros((HR * 128,), jnp.float32)
    zrow = jnp.zeros((128, 16), jnp.float32)

    hists = _deg_kernel(dst3, zeros1d)
    deg2d = _k0(hists.reshape(NC * NS, HR, 128))
    h = deg2d.reshape(-1)[:N_PAD].reshape(N_PAD, 1)

    dis, y1 = _k1(h, pos_p)

    (s1,) = _prop1(src3, dst3, zrow, y1)
    y2 = _k2(dis, s1[0], s1[1], y1, w1p, b1r)

    s2 = _prop4(src3, dst3, zrow, *y2)
    args3 = [x for g in range(4) for x in (s2[g][0], s2[g][1], y2[g])]
    y3 = _k3(dis, *args3, W2, b2r)

    s3 = _prop4(src3, dst3, zrow, *y3)
    args4 = [x for g in range(4) for x in (s3[g][0], s3[g][1], y3[g])]
    out = _k4(dis, *args4, W3, b3r)
    return out[:N]


# speedup vs baseline: 15.3446x; 1.5333x over previous
import functools

import jax
import jax.numpy as jnp
from jax import lax
from jax.experimental import pallas as pl
from jax.experimental.pallas import tpu as pltpu
from jax.experimental.pallas import tpu_sc as plsc

N = 50000
E = 800000
N_PAD = 51200
ROWS_PER_TILE = N_PAD // 16
DRAIN_CH = ROWS_PER_TILE // 128
NC, NS = 2, 16
J = 196
E_PAD = NC * NS * J * 128
HR = 512
BLK = 512
GRID = N_PAD // BLK

_mesh = plsc.VectorSubcoreMesh(
    core_axis_name="c", subcore_axis_name="s", num_cores=NC, num_subcores=NS)


@functools.partial(
    pl.kernel,
    out_type=jax.ShapeDtypeStruct((NC * NS, HR * 128), jnp.float32),
    mesh=_mesh,
    scratch_types=[
        pltpu.VMEM((J, 128), jnp.int32),
        pltpu.VMEM((HR * 128,), jnp.float32),
    ],
    compiler_params=pltpu.CompilerParams(needs_layout_passes=False),
)
def _deg_kernel(dst3, zeros1d, histout, dstv, hist):
    c = lax.axis_index("c")
    s = lax.axis_index("s")
    wid = c * NS + s
    pltpu.sync_copy(dst3.at[wid], dstv)
    pltpu.sync_copy(zeros1d, hist)
    ones16 = jnp.full((16,), 1.0, jnp.float32)

    def hbody(j, carry):
        for l in range(8):
            idx = dstv[j, pl.ds(l * 16, 16)]
            plsc.addupdate_scatter(hist, [idx], ones16)
        return carry

    lax.fori_loop(0, J, hbody, 0)
    pltpu.sync_copy(hist, histout.at[wid])


def _make_prop(num_groups, cg):
    D = 4

    @functools.partial(
        pl.kernel,
        out_type=[jax.ShapeDtypeStruct((NC, N_PAD, cg), jnp.float32)
                  for _ in range(num_groups)],
        mesh=_mesh,
        scratch_types=[
            pltpu.VMEM((J, 128), jnp.int32),
            pltpu.VMEM((J, 128), jnp.int32),
            pltpu.VMEM((D, 128, cg), jnp.float32),
            pltpu.VMEM_SHARED((N_PAD, cg), jnp.float32),
        ] + [pltpu.SemaphoreType.DMA] * (2 * D),
        compiler_params=pltpu.CompilerParams(
            needs_layout_passes=False, use_tc_tiling_on_sc=False),
    )
    def prop(src3, dst3, zbig, *rest):
        ys = rest[:num_groups]
        outs = rest[num_groups:2 * num_groups]
        srcv, dstv, gbuf, acc = rest[2 * num_groups:2 * num_groups + 4]
        gsem = rest[2 * num_groups + 4:2 * num_groups + 4 + D]
        ssem = rest[2 * num_groups + 4 + D:]
        c = lax.axis_index("c")
        s = lax.axis_index("s")
        wid = c * NS + s
        base = s * ROWS_PER_TILE
        pltpu.sync_copy(src3.at[wid], srcv)
        pltpu.sync_copy(dst3.at[wid], dstv)
        for g in range(num_groups):
            y = ys[g]
            out = outs[g]
            pltpu.sync_copy(zbig, acc.at[pl.ds(base, ROWS_PER_TILE)])
            plsc.subcore_barrier()
            for b in range(D):
                pltpu.async_copy(y.at[srcv.at[b]], gbuf.at[b], gsem[b])

            def ebody(t, carry):
                for b in range(D):
                    j = t * D + b
                    pltpu.make_async_copy(
                        y.at[srcv.at[j]], gbuf.at[b], gsem[b]).wait()
                    pltpu.async_copy(
                        gbuf.at[b], acc.at[dstv.at[j]], ssem[b], add=True).wait()
                    pltpu.async_copy(y.at[srcv.at[j + D]], gbuf.at[b], gsem[b])
                return carry

            lax.fori_loop(0, J // D - 1, ebody, 0)
            for b in range(D):
                j = J - D + b
                pltpu.make_async_copy(
                    y.at[srcv.at[j]], gbuf.at[b], gsem[b]).wait()
                pltpu.async_copy(
                    gbuf.at[b], acc.at[dstv.at[j]], ssem[b], add=True).wait()
            plsc.subcore_barrier()
            pltpu.sync_copy(acc.at[pl.ds(base, ROWS_PER_TILE)],
                            out.at[c, pl.ds(base, ROWS_PER_TILE)])

    return prop


_prop1 = _make_prop(1, 16)
_prop4 = _make_prop(4, 16)


def _row_spec(ch):
    return pl.BlockSpec((BLK, ch), lambda i: (i, 0))


def _full_spec(r, ch):
    return pl.BlockSpec((r, ch), lambda i: (0, 0))


def _k0_body(hs, degr):
    degr[...] = jnp.sum(hs[...], axis=0)


def _k1_body(h, posr, disr, y1r):
    deg = h[...] + 1.0
    dis = lax.rsqrt(deg)
    disr[...] = dis
    y1r[...] = dis * posr[...]


def _k2_body(disr, s0, s1, y1, w, b, *youts):
    dis = disr[...]
    z = dis * (s0[...] + s1[...] + y1[...])
    h = jnp.maximum(
        jnp.dot(z, w[...], preferred_element_type=jnp.float32) + b[...], 0.0)
    y2 = dis * h
    for g in range(4):
        youts[g][...] = y2[:, g * 16:(g + 1) * 16]


def _gather_z(disr, srefs):
    dis = disr[...]
    zs = [dis * (srefs[3 * g][...] + srefs[3 * g + 1][...] + srefs[3 * g + 2][...])
          for g in range(4)]
    return dis, jnp.concatenate(zs, axis=1)


def _k3_body(disr, *rest):
    srefs, (w, b) = rest[:12], rest[12:14]
    youts = rest[14:]
    dis, z = _gather_z(disr, srefs)
    h = jnp.maximum(
        jnp.dot(z, w[...], preferred_element_type=jnp.float32) + b[...], 0.0)
    y3 = dis * h
    for g in range(4):
        youts[g][...] = y3[:, g * 16:(g + 1) * 16]


def _k4_body(disr, *rest):
    srefs, (w, b), outr = rest[:12], rest[12:14], rest[14]
    _, z = _gather_z(disr, srefs)
    t = jnp.dot(z, w[...], preferred_element_type=jnp.float32) + b[...]
    outr[...] = 1.0 / (1.0 + jnp.exp(-t))


def _shape(ch):
    return jax.ShapeDtypeStruct((N_PAD, ch), jnp.float32)


_k0 = pl.pallas_call(
    _k0_body, grid=(4,),
    in_specs=[pl.BlockSpec((NC * NS, 128, 128), lambda i: (0, i, 0))],
    out_specs=pl.BlockSpec((128, 128), lambda i: (i, 0)),
    out_shape=jax.ShapeDtypeStruct((HR, 128), jnp.float32))

_k1 = pl.pallas_call(
    _k1_body, grid=(GRID,),
    in_specs=[_row_spec(1), _row_spec(16)],
    out_specs=[_row_spec(1), _row_spec(16)],
    out_shape=[_shape(1), _shape(16)])

_k2 = pl.pallas_call(
    _k2_body, grid=(GRID,),
    in_specs=[_row_spec(1), _row_spec(16), _row_spec(16), _row_spec(16),
              _full_spec(16, 64), _full_spec(1, 64)],
    out_specs=[_row_spec(16)] * 4,
    out_shape=[_shape(16)] * 4)

_k3 = pl.pallas_call(
    _k3_body, grid=(GRID,),
    in_specs=[_row_spec(1)] + [_row_spec(16)] * 12 +
             [_full_spec(64, 64), _full_spec(1, 64)],
    out_specs=[_row_spec(16)] * 4,
    out_shape=[_shape(16)] * 4)

_k4 = pl.pallas_call(
    _k4_body, grid=(GRID,),
    in_specs=[_row_spec(1)] + [_row_spec(16)] * 12 +
             [_full_spec(64, 68), _full_spec(1, 68)],
    out_specs=_row_spec(68),
    out_shape=_shape(68))


def kernel(pos, edge_index, W1, b1, W2, b2, W3, b3):
    src = edge_index[0]
    dst = edge_index[1]
    pad = jnp.full((E_PAD - E,), N, jnp.int32)
    src3 = jnp.concatenate([src, pad]).reshape(NC * NS, J, 128)
    dst3 = jnp.concatenate([dst, pad]).reshape(NC * NS, J, 128)

    pos_p = jnp.zeros((N_PAD, 16), jnp.float32).at[:N, :3].set(pos)
    w1p = jnp.zeros((16, 64), jnp.float32).at[:3].set(W1)
    b1r = b1.reshape(1, 64)
    b2r = b2.reshape(1, 64)
    b3r = b3.reshape(1, 68)

    zeros1d = jnp.zeros((HR * 128,), jnp.float32)
    zrow = jnp.zeros((ROWS_PER_TILE, 16), jnp.float32)

    hists = _deg_kernel(dst3, zeros1d)
    deg2d = _k0(hists.reshape(NC * NS, HR, 128))
    h = deg2d.reshape(-1)[:N_PAD].reshape(N_PAD, 1)

    dis, y1 = _k1(h, pos_p)

    (s1,) = _prop1(src3, dst3, zrow, y1)
    y2 = _k2(dis, s1[0], s1[1], y1, w1p, b1r)

    s2 = _prop4(src3, dst3, zrow, *y2)
    args3 = [x for g in range(4) for x in (s2[g][0], s2[g][1], y2[g])]
    y3 = _k3(dis, *args3, W2, b2r)

    s3 = _prop4(src3, dst3, zrow, *y3)
    args4 = [x for g in range(4) for x in (s3[g][0], s3[g][1], y3[g])]
    out = _k4(dis, *args4, W3, b3r)
    return out[:N]


# speedup vs baseline: 20.2724x; 1.3211x over previous
import functools

import jax
import jax.numpy as jnp
from jax import lax
from jax.experimental import pallas as pl
from jax.experimental.pallas import tpu as pltpu
from jax.experimental.pallas import tpu_sc as plsc

N = 50000
E = 800000
N_PAD = 51200
ROWS_PER_TILE = N_PAD // 16
NC, NS = 2, 16
J = 196
E_PAD = NC * NS * J * 128
BLK = 512
GRID = N_PAD // BLK

_mesh = plsc.VectorSubcoreMesh(
    core_axis_name="c", subcore_axis_name="s", num_cores=NC, num_subcores=NS)


HR2 = N_PAD // 128


@functools.partial(
    pl.kernel,
    out_type=[
        jax.ShapeDtypeStruct((HR2, 128), jnp.float32),
        jax.ShapeDtypeStruct((N_PAD, 16), jnp.float32),
    ],
    mesh=_mesh,
    scratch_types=[
        pltpu.VMEM((J, 128), jnp.int32),
        pltpu.VMEM((HR2, 128), jnp.float32),
        pltpu.VMEM((4, 100), jnp.int32),
        pltpu.VMEM((25, 128), jnp.float32),
        pltpu.VMEM((128, 16), jnp.float32),
        pltpu.VMEM((128, 16), jnp.float32),
        pltpu.VMEM_SHARED((HR2, 128), jnp.float32),
    ],
    compiler_params=pltpu.CompilerParams(
        needs_layout_passes=False, use_tc_tiling_on_sc=False),
)
def _front(dst3, pos_p, iota4,
           dis_out, u_out,
           dstv, hist, iov, degb, pbuf, ubuf, acc_h):
    c = lax.axis_index("c")
    s = lax.axis_index("s")
    base = s * ROWS_PER_TILE
    pltpu.sync_copy(iota4, iov)
    zeros16 = jnp.zeros((16,), jnp.float32)

    def zbody(r, carry):
        for l in range(8):
            hist[r, pl.ds(l * 16, 16)] = zeros16
        return carry

    lax.fori_loop(0, HR2, zbody, 0)
    pltpu.sync_copy(hist.at[pl.ds(s * 25, 25)], acc_h.at[pl.ds(s * 25, 25)])
    ones16 = jnp.full((16,), 1.0, jnp.float32)

    def hbody(j, carry):
        for l in range(8):
            idx = dstv[j, pl.ds(l * 16, 16)]
            row = lax.shift_right_logical(idx, 7)
            col = lax.bitwise_and(idx, 127)
            plsc.addupdate_scatter(hist, [row, col], ones16)
        return carry

    pltpu.sync_copy(dst3.at[s], dstv)
    lax.fori_loop(0, J, hbody, 0)
    pltpu.sync_copy(dst3.at[s + 16], dstv)
    lax.fori_loop(0, J, hbody, 0)
    plsc.subcore_barrier()
    for q in range(4):
        pltpu.sync_copy(hist.at[pl.ds(q * 100, 100)],
                        acc_h.at[iov.at[q]], add=True)
    plsc.subcore_barrier()

    pltpu.sync_copy(acc_h.at[pl.ds(s * 25, 25)], degb)

    def dbody(r, carry):
        for l in range(8):
            v = degb[r, pl.ds(l * 16, 16)] + 1.0
            i = jnp.int32(0x5F3759DF) - lax.shift_right_logical(
                plsc.bitcast(v, jnp.int32), 1)
            y = plsc.bitcast(i, jnp.float32)
            for _ in range(3):
                y = y * (1.5 - 0.5 * v * y * y)
            degb[r, pl.ds(l * 16, 16)] = y
        return carry

    lax.fori_loop(0, 25, dbody, 0)
    pltpu.sync_copy(degb, dis_out.at[pl.ds(s * 25, 25)])

    def ubody(q, carry):
        pltpu.sync_copy(pos_p.at[pl.ds(base + q * 128, 128)], pbuf)
        for r16 in range(8):
            dv = degb[q, pl.ds(r16 * 16, 16)]
            for m in range(16):
                r = r16 * 16 + m
                ubuf[r] = pbuf[r] * dv[m]
        pltpu.sync_copy(ubuf, u_out.at[pl.ds(base + q * 128, 128)])
        return carry

    lax.fori_loop(0, 25, ubody, 0)


def _make_prop(num_groups, cg):
    D = 14
    K = 1
    JS = J // K
    assert J % K == 0 and JS % D == 0

    @functools.partial(
        pl.kernel,
        out_type=[jax.ShapeDtypeStruct((NC, N_PAD, cg), jnp.float32)
                  for _ in range(num_groups)],
        mesh=_mesh,
        scratch_types=[
            pltpu.VMEM((JS, K * 128), jnp.int32),
            pltpu.VMEM((JS, K * 128), jnp.int32),
            pltpu.VMEM((D, K * 128, cg), jnp.float32),
            pltpu.VMEM_SHARED((N_PAD, cg), jnp.float32),
        ] + [pltpu.SemaphoreType.DMA] * (2 * D),
        compiler_params=pltpu.CompilerParams(
            needs_layout_passes=False, use_tc_tiling_on_sc=False),
    )
    def prop(src4, dst4, zbig, *rest):
        ys = rest[:num_groups]
        outs = rest[num_groups:2 * num_groups]
        srcv, dstv, gbuf, acc = rest[2 * num_groups:2 * num_groups + 4]
        gsem = rest[2 * num_groups + 4:2 * num_groups + 4 + D]
        ssem = rest[2 * num_groups + 4 + D:]
        c = lax.axis_index("c")
        s = lax.axis_index("s")
        wid = c * NS + s
        base = s * ROWS_PER_TILE
        pltpu.sync_copy(src4.at[wid], srcv)
        pltpu.sync_copy(dst4.at[wid], dstv)
        for g in range(num_groups):
            y = ys[g]
            out = outs[g]
            pltpu.sync_copy(zbig, acc.at[pl.ds(base, ROWS_PER_TILE)])
            plsc.subcore_barrier()
            for b in range(D):
                pltpu.async_copy(y.at[srcv.at[b]], gbuf.at[b], gsem[b])

            def ebody(t, carry):
                for b in range(D):
                    j = t * D + b
                    pltpu.make_async_copy(
                        y.at[srcv.at[j]], gbuf.at[b], gsem[b]).wait()
                    pltpu.async_copy(
                        gbuf.at[b], acc.at[dstv.at[j]], ssem[b], add=True).wait()
                    pltpu.async_copy(y.at[srcv.at[j + D]], gbuf.at[b], gsem[b])
                return carry

            lax.fori_loop(0, JS // D - 1, ebody, 0)
            for b in range(D):
                j = JS - D + b
                pltpu.make_async_copy(
                    y.at[srcv.at[j]], gbuf.at[b], gsem[b]).wait()
                pltpu.async_copy(
                    gbuf.at[b], acc.at[dstv.at[j]], ssem[b], add=True).wait()
            plsc.subcore_barrier()
            pltpu.sync_copy(acc.at[pl.ds(base, ROWS_PER_TILE)],
                            out.at[c, pl.ds(base, ROWS_PER_TILE)])

    return prop


_prop1 = _make_prop(1, 16)
_prop4 = _make_prop(4, 16)


def _row_spec(ch):
    return pl.BlockSpec((BLK, ch), lambda i: (i, 0))


def _full_spec(r, ch):
    return pl.BlockSpec((r, ch), lambda i: (0, 0))


def _k2_body(disr, sr, y1, w, b, *youts):
    dis = disr[...]
    z = dis * (sr[0] + sr[1] + y1[...])
    h = jnp.maximum(
        jnp.dot(z, w[...], preferred_element_type=jnp.float32) + b[...], 0.0)
    y2 = dis * h
    for g in range(4):
        youts[g][...] = y2[:, g * 16:(g + 1) * 16]


def _gather_z(disr, srefs):
    dis = disr[...]
    zs = [dis * (srefs[2 * g][0] + srefs[2 * g][1] + srefs[2 * g + 1][...])
          for g in range(4)]
    return dis, jnp.concatenate(zs, axis=1)


def _k3_body(disr, *rest):
    srefs, (w, b) = rest[:8], rest[8:10]
    youts = rest[10:]
    dis, z = _gather_z(disr, srefs)
    h = jnp.maximum(
        jnp.dot(z, w[...], preferred_element_type=jnp.float32) + b[...], 0.0)
    y3 = dis * h
    for g in range(4):
        youts[g][...] = y3[:, g * 16:(g + 1) * 16]


def _k4_body(disr, *rest):
    srefs, (w, b), outr = rest[:8], rest[8:10], rest[10]
    _, z = _gather_z(disr, srefs)
    t = jnp.dot(z, w[...], preferred_element_type=jnp.float32) + b[...]
    outr[...] = 1.0 / (1.0 + jnp.exp(-t))


def _shape(ch):
    return jax.ShapeDtypeStruct((N_PAD, ch), jnp.float32)


_s_spec = pl.BlockSpec((NC, BLK, 16), lambda i: (0, i, 0))

_k2 = pl.pallas_call(
    _k2_body, grid=(GRID,),
    in_specs=[_row_spec(1), _s_spec, _row_spec(16),
              _full_spec(16, 64), _full_spec(1, 64)],
    out_specs=[_row_spec(16)] * 4,
    out_shape=[_shape(16)] * 4)

_k3 = pl.pallas_call(
    _k3_body, grid=(GRID,),
    in_specs=[_row_spec(1)] + [_s_spec, _row_spec(16)] * 4 +
             [_full_spec(64, 64), _full_spec(1, 64)],
    out_specs=[_row_spec(16)] * 4,
    out_shape=[_shape(16)] * 4)

_k4 = pl.pallas_call(
    _k4_body, grid=(GRID,),
    in_specs=[_row_spec(1)] + [_s_spec, _row_spec(16)] * 4 +
             [_full_spec(64, 68), _full_spec(1, 68)],
    out_specs=_row_spec(68),
    out_shape=_shape(68))


def kernel(pos, edge_index, W1, b1, W2, b2, W3, b3):
    src = edge_index[0]
    dst = edge_index[1]
    pad = jnp.full((E_PAD - E,), N, jnp.int32)
    src3 = jnp.concatenate([src, pad]).reshape(NC * NS, J, 128)
    dst3 = jnp.concatenate([dst, pad]).reshape(NC * NS, J, 128)
    src4 = src3
    dst4 = dst3

    pos_p = jnp.zeros((N_PAD, 16), jnp.float32).at[:N, :3].set(pos)
    w1p = jnp.zeros((16, 64), jnp.float32).at[:3].set(W1)
    b1r = b1.reshape(1, 64)
    b2r = b2.reshape(1, 64)
    b3r = b3.reshape(1, 68)

    iota4 = jnp.arange(HR2, dtype=jnp.int32).reshape(4, 100)
    zrow = jnp.zeros((ROWS_PER_TILE, 16), jnp.float32)

    dis2d, y1 = _front(dst3, pos_p, iota4)
    dis = dis2d.reshape(N_PAD, 1)

    (s1,) = _prop1(src4, dst4, zrow, y1)
    y2 = _k2(dis, s1, y1, w1p, b1r)

    s2 = _prop4(src4, dst4, zrow, *y2)
    args3 = [x for g in range(4) for x in (s2[g], y2[g])]
    y3 = _k3(dis, *args3, W2, b2r)

    s3 = _prop4(src4, dst4, zrow, *y3)
    args4 = [x for g in range(4) for x in (s3[g], y3[g])]
    out = _k4(dis, *args4, W3, b3r)
    return out[:N]
